# baseline (speedup 1.0000x reference)
import jax
import jax.numpy as jnp
from jax.experimental import pallas as pl
from jax.experimental.pallas import tpu as pltpu

_VMEM_LIMIT = 61 * 1024 * 1024


def _tapsD(r):
    return (0, 1) if r == 0 else (1, 2)


def _mlp_fc5_kernel(c_ref, vt_ref,
                    w1c_ref, b1c_ref, w2c_ref, b2c_ref,
                    w1vt_ref, b1vt_ref, w2vt_ref, b2vt_ref,
                    w3c_ref, w3vt_ref, b3_ref, w4_ref, b4_ref,
                    w5_ref, b5_ref, o_ref):
    def ff(x, w_ref, b_ref):
        y = jnp.dot(x, w_ref[...], preferred_element_type=jnp.float32)
        return jnp.maximum(y + b_ref[...], 0.0).astype(jnp.bfloat16)

    cb = ff(ff(c_ref[...], w1c_ref, b1c_ref), w2c_ref, b2c_ref)
    vtb = ff(ff(vt_ref[...], w1vt_ref, b1vt_ref), w2vt_ref, b2vt_ref)
    h3 = (jnp.dot(cb, w3c_ref[...], preferred_element_type=jnp.float32)
          + jnp.dot(vtb, w3vt_ref[...], preferred_element_type=jnp.float32)
          + b3_ref[...])
    h3 = jnp.maximum(h3, 0.0).astype(jnp.bfloat16)
    h4 = ff(h3, w4_ref, b4_ref)
    o_ref[...] = ff(h4, w5_ref, b5_ref)


def _run_mlp_fc5(c, vt, wlist, bt=64):
    B = c.shape[0]
    args = [c, vt] + wlist

    def spec(a):
        nd = a.ndim
        return pl.BlockSpec(a.shape, lambda i, nd=nd: (0,) * nd)

    in_specs = [pl.BlockSpec((bt, c.shape[1]), lambda i: (i, 0)),
                pl.BlockSpec((bt, vt.shape[1]), lambda i: (i, 0))]
    in_specs += [spec(a) for a in wlist]
    return pl.pallas_call(
        _mlp_fc5_kernel,
        out_shape=jax.ShapeDtypeStruct((B, 16384), jnp.bfloat16),
        grid_spec=pltpu.PrefetchScalarGridSpec(
            num_scalar_prefetch=0,
            grid=(B // bt,),
            in_specs=in_specs,
            out_specs=pl.BlockSpec((bt, 16384), lambda i: (i, 0)),
        ),
        compiler_params=pltpu.CompilerParams(
            dimension_semantics=("parallel",),
            vmem_limit_bytes=_VMEM_LIMIT,
        ),
    )(*args)


def _zero_borders(S):
    bb, hp, wp, ch = S.shape
    S[:, 0:1, :, :] = jnp.zeros((bb, 1, wp, ch), S.dtype)
    S[:, hp - 1:hp, :, :] = jnp.zeros((bb, 1, wp, ch), S.dtype)
    S[:, :, 0:1, :] = jnp.zeros((bb, hp, 1, ch), S.dtype)
    S[:, :, wp - 1:wp, :] = jnp.zeros((bb, hp, 1, ch), S.dtype)


def _patch(S, taps, h, w, bb, cin):
    parts = [S[:, dy:dy + h, dx:dx + w, :] for (dy, dx) in taps]
    return jnp.concatenate(parts, axis=-1).reshape(bb * h * w, len(taps) * cin)


def _sx(S, h, w):
    return jnp.concatenate([S[:, :, dx:dx + w, :] for dx in range(3)], axis=-1)


def _rows_interleave(top, bot, bb, h, w2, cp):
    z = jnp.concatenate([top[:, :, None, :, :], bot[:, :, None, :, :]], axis=2)
    return z.reshape(bb, 2 * h, w2, cp)


def _decoder_kernel(x_ref,
                    u1w_ref, u1b_ref, c1w_ref, c1b_ref,
                    u2w_ref, u2b_ref, c2w_ref, c2b_ref,
                    u3w_ref, u3b_ref, c3w_ref, c3b_ref,
                    hw_ref, hb_ref,
                    o_ref,
                    S0, S1, S1b, S2, S2b, S3, S3b, *, bb):
    f32 = jnp.float32
    for S in (S0, S1, S1b, S2, S2b, S3, S3b):
        _zero_borders(S)

    S0[:, 1:9, 1:9, :] = x_ref[...]

    outs = []
    for ry in (0, 1):
        for rx in (0, 1):
            p = 2 * ry + rx
            taps = [(dy, dx) for dy in _tapsD(ry) for dx in _tapsD(rx)]
            pt = _patch(S0, taps, 8, 8, bb, 256)
            y = jnp.dot(pt, u1w_ref[p], preferred_element_type=f32)
            y = jnp.maximum(y + u1b_ref[p], 0.0).astype(jnp.bfloat16)
            outs.append(y.reshape(bb, 8, 8, 256))
    top = jnp.concatenate([outs[0][:, :, :, None, :], outs[1][:, :, :, None, :]],
                          axis=3).reshape(bb, 8, 16, 256)
    bot = jnp.concatenate([outs[2][:, :, :, None, :], outs[3][:, :, :, None, :]],
                          axis=3).reshape(bb, 8, 16, 256)
    S1[:, 1:17, 1:17, :] = _rows_interleave(top, bot, bb, 8, 16, 256)

    def full_patch(sx, h, w, kc):
        pt = jnp.concatenate([sx[:, dy:dy + h] for dy in range(3)], axis=-1)
        return pt.reshape(bb * h * w, kc)

    sx = _sx(S1, 16, 16)
    acc = jnp.dot(full_patch(sx, 16, 16, 2304), c1w_ref[...],
                  preferred_element_type=f32)
    acc = jnp.maximum(acc + c1b_ref[...], 0.0).astype(jnp.bfloat16)
    S1b[:, 1:17, 1:17, :] = acc.reshape(bb, 16, 16, 256)

    sx = _sx(S1b, 16, 16)
    rows = []
    for ry in (0, 1):
        d0, d1 = _tapsD(ry)
        pt = jnp.concatenate([sx[:, d0:d0 + 16], sx[:, d1:d1 + 16]],
                             axis=-1).reshape(bb * 256, 1536)
        acc = jnp.dot(pt, u2w_ref[ry], preferred_element_type=f32)
        y = jnp.maximum(acc + u2b_ref[ry], 0.0).astype(jnp.bfloat16)
        rows.append(y.reshape(bb, 16, 16, 2, 128).reshape(bb, 16, 32, 128))
    S2[:, 1:33, 1:33, :] = _rows_interleave(rows[0], rows[1], bb, 16, 32, 128)

    sx = _sx(S2, 32, 32)
    acc = jnp.dot(full_patch(sx, 32, 32, 1152), c2w_ref[...],
                  preferred_element_type=f32)
    acc = jnp.maximum(acc + c2b_ref[...], 0.0).astype(jnp.bfloat16)
    S2b[:, 1:33, 1:33, :] = acc.reshape(bb, 32, 32, 128)

    sx = _sx(S2b, 32, 32)
    acc = jnp.dot(full_patch(sx, 32, 32, 1152), u3w_ref[...],
                  preferred_element_type=f32)
    y = jnp.maximum(acc + u3b_ref[...], 0.0).astype(jnp.bfloat16)
    y = y.reshape(bb, 32, 32, 2, 2, 64)
    top = y[:, :, :, 0].reshape(bb, 32, 64, 64)
    bot = y[:, :, :, 1].reshape(bb, 32, 64, 64)
    S3[:, 1:65, 1:65, :] = _rows_interleave(top, bot, bb, 32, 64, 64)

    sx = _sx(S3, 64, 64)
    acc = jnp.dot(full_patch(sx, 64, 64, 576), c3w_ref[...],
                  preferred_element_type=f32)
    acc = jnp.maximum(acc + c3b_ref[...], 0.0).astype(jnp.bfloat16)
    S3b[:, 1:65, 1:65, :] = acc.reshape(bb, 64, 64, 64)

    sx = _sx(S3b, 64, 64)
    acc = jnp.dot(full_patch(sx, 64, 64, 576), hw_ref[...],
                  preferred_element_type=f32)
    acc = acc + hb_ref[...]
    lane = jax.lax.broadcasted_iota(jnp.int32, acc.shape, 1)
    acc = jnp.where(lane % 4 == 3, jax.nn.sigmoid(acc), acc)
    o_ref[...] = acc.reshape(bb, 64, 64, 16)


def _run_decoder(x, wlist, bb=4):
    B = x.shape[0]

    def spec(a):
        nd = a.ndim
        return pl.BlockSpec(a.shape, lambda i, nd=nd: (0,) * nd)

    bf = jnp.bfloat16
    scratch = [
        pltpu.VMEM((bb, 10, 10, 256), bf),
        pltpu.VMEM((bb, 18, 18, 256), bf),
        pltpu.VMEM((bb, 18, 18, 256), bf),
        pltpu.VMEM((bb, 34, 34, 128), bf),
        pltpu.VMEM((bb, 34, 34, 128), bf),
        pltpu.VMEM((bb, 66, 66, 64), bf),
        pltpu.VMEM((bb, 66, 66, 64), bf),
    ]
    import functools
    return pl.pallas_call(
        functools.partial(_decoder_kernel, bb=bb),
        out_shape=jax.ShapeDtypeStruct((B, 64, 64, 16), jnp.float32),
        grid_spec=pltpu.PrefetchScalarGridSpec(
            num_scalar_prefetch=0,
            grid=(B // bb,),
            in_specs=[pl.BlockSpec((bb, 8, 8, 256), lambda i: (i, 0, 0, 0))]
                     + [spec(a) for a in wlist],
            out_specs=pl.BlockSpec((bb, 64, 64, 16), lambda i: (i, 0, 0, 0)),
            scratch_shapes=scratch,
        ),
        compiler_params=pltpu.CompilerParams(
            dimension_semantics=("parallel",),
            vmem_limit_bytes=_VMEM_LIMIT,
        ),
    )(x, *wlist)


def _prep_up_parity(w, b, cp):
    ws, bs = [], []
    for ry in (0, 1):
        for rx in (0, 1):
            p = 2 * ry + rx
            blocks = [w[3 * dy + dx, :, p * cp:(p + 1) * cp]
                      for dy in _tapsD(ry) for dx in _tapsD(rx)]
            ws.append(jnp.concatenate(blocks, 0))
            bs.append(b[:, p * cp:(p + 1) * cp])
    return jnp.stack(ws), jnp.stack(bs)


def _prep_up_pair(w, b, cp):
    ws, bs = [], []
    for ry in (0, 1):
        rows = []
        for dy in _tapsD(ry):
            for dx in range(3):
                cols = []
                for rx in (0, 1):
                    blk = w[3 * dy + dx, :, (2 * ry + rx) * cp:(2 * ry + rx + 1) * cp]
                    if dx not in _tapsD(rx):
                        blk = jnp.zeros_like(blk)
                    cols.append(blk)
                rows.append(jnp.concatenate(cols, 1))
        ws.append(jnp.concatenate(rows, 0))
        bs.append(jnp.concatenate([b[:, 2 * ry * cp:(2 * ry + 1) * cp],
                                   b[:, (2 * ry + 1) * cp:(2 * ry + 2) * cp]], 1))
    return jnp.stack(ws), jnp.stack(bs)


def _prep_up_quad(w, b, cp_in, cp_out):
    rows = []
    for tap in range(9):
        dy, dx = tap // 3, tap % 3
        cols = []
        for ry in (0, 1):
            for rx in (0, 1):
                blk = w[tap, :, (2 * ry + rx) * cp_in:(2 * ry + rx) * cp_in + cp_out]
                if (dy not in _tapsD(ry)) or (dx not in _tapsD(rx)):
                    blk = jnp.zeros_like(blk)
                cols.append(blk)
        rows.append(jnp.concatenate(cols, 1))
    wq = jnp.concatenate(rows, 0)
    bq = jnp.concatenate([b[:, p * cp_in:p * cp_in + cp_out] for p in range(4)], 1)
    return wq, bq


def kernel(mlp_00, mlp_01, mlp_02, mlp_03, mlp_04, mlp_05, mlp_06, mlp_07,
           mlp_08, mlp_09, mlp_10, mlp_11, mlp_12, mlp_13, mlp_14, mlp_15,
           fc5_w, fc5_b,
           upconv1_w, upconv1_b, conv1_1_w, conv1_1_b,
           upconv2_w, upconv2_b, conv2_1_w, conv2_1_b,
           upconv3_w, upconv3_b, conv3_1_w, conv3_1_b,
           head_w, head_b,
           c, v, t):
    bf = jnp.bfloat16
    B = c.shape[0]

    w1c, b1c, w2c, b2c = mlp_00, mlp_01, mlp_02, mlp_03
    w1v, b1v, w2v, b2v = mlp_04, mlp_05, mlp_06, mlp_07
    w1t, b1t, w2t, b2t = mlp_08, mlp_09, mlp_10, mlp_11
    w3, b3, w4, b4 = mlp_12, mlp_13, mlp_14, mlp_15

    z44 = jnp.zeros((4, 512), bf)
    z12 = jnp.zeros((12, 512), bf)
    w1vt = jnp.concatenate(
        [jnp.concatenate([w1v, z44], 1), jnp.concatenate([z12, w1t], 1)], 0)
    b1vt = jnp.concatenate([b1v, b1t], 1)
    z5 = jnp.zeros((512, 512), bf)
    w2vt = jnp.concatenate(
        [jnp.concatenate([w2v, z5], 1), jnp.concatenate([z5, w2t], 1)], 0)
    b2vt = jnp.concatenate([b2v, b2t], 1)
    w3c, w3vt = w3[0:512], w3[512:1536]

    vt = jnp.concatenate([v, t], 1).astype(bf)
    mlp_wlist = [w1c, b1c, w2c, b2c, w1vt, b1vt, w2vt, b2vt,
                 w3c, w3vt, b3, w4, b4, fc5_w, fc5_b]
    bt = 128 if B % 128 == 0 else B
    x = _run_mlp_fc5(c.astype(bf), vt, mlp_wlist, bt=bt)
    x = x.reshape(B, 8, 8, 256)

    def rowform(w9):
        return jnp.stack([jnp.concatenate([w9[3 * dy + dx] for dx in range(3)], 0)
                          for dy in range(3)])

    u1w, u1b = _prep_up_parity(upconv1_w, upconv1_b, 256)
    c1w = rowform(conv1_1_w).reshape(2304, 256)
    u2w, u2b = _prep_up_pair(upconv2_w, upconv2_b, 128)
    c2w = rowform(conv2_1_w).reshape(1152, 128)
    u3w, u3b = _prep_up_quad(upconv3_w, upconv3_b, 128, 64)
    c3w = rowform(conv3_1_w[:, :64, :64]).reshape(576, 64)
    c3b = conv3_1_b[:, :64]
    hw16 = jnp.concatenate([head_w[:, :64, p * 32:p * 32 + 4] for p in range(4)], 2)
    hw = rowform(hw16).reshape(576, 16)
    hb = jnp.concatenate([head_b[:, p * 32:p * 32 + 4] for p in range(4)], 1)

    dec_wlist = [u1w, u1b, c1w, conv1_1_b, u2w, u2b, c2w, conv2_1_b,
                 u3w, u3b, c3w, c3b, hw, hb]
    bb = 1
    head = _run_decoder(x, dec_wlist, bb=bb)

    r = head.reshape(B, 64, 64, 2, 2, 4)
    r = r.transpose(0, 5, 1, 3, 2, 4).reshape(B, 4, 128, 128)
    return r[:, 0:3], r[:, 3:4]


# speedup vs baseline: 1.0677x; 1.0677x over previous
import jax
import jax.numpy as jnp
from jax.experimental import pallas as pl
from jax.experimental.pallas import tpu as pltpu

_VMEM_LIMIT = 61 * 1024 * 1024


def _tapsD(r):
    return (0, 1) if r == 0 else (1, 2)


def _mlp_fc5_kernel(c_ref, vt_ref,
                    w1c_ref, b1c_ref, w2c_ref, b2c_ref,
                    w1vt_ref, b1vt_ref, w2vt_ref, b2vt_ref,
                    w3c_ref, w3vt_ref, b3_ref, w4_ref, b4_ref,
                    w5_ref, b5_ref, o_ref):
    def ff(x, w_ref, b_ref):
        y = jnp.dot(x, w_ref[...], preferred_element_type=jnp.float32)
        return jnp.maximum(y + b_ref[...], 0.0).astype(jnp.bfloat16)

    cb = ff(ff(c_ref[...], w1c_ref, b1c_ref), w2c_ref, b2c_ref)
    vtb = ff(ff(vt_ref[...], w1vt_ref, b1vt_ref), w2vt_ref, b2vt_ref)
    h3 = (jnp.dot(cb, w3c_ref[...], preferred_element_type=jnp.float32)
          + jnp.dot(vtb, w3vt_ref[...], preferred_element_type=jnp.float32)
          + b3_ref[...])
    h3 = jnp.maximum(h3, 0.0).astype(jnp.bfloat16)
    h4 = ff(h3, w4_ref, b4_ref)
    o_ref[...] = ff(h4, w5_ref, b5_ref)


def _run_mlp_fc5(c, vt, wlist, bt=64):
    B = c.shape[0]
    args = [c, vt] + wlist

    def spec(a):
        nd = a.ndim
        return pl.BlockSpec(a.shape, lambda i, nd=nd: (0,) * nd)

    in_specs = [pl.BlockSpec((bt, c.shape[1]), lambda i: (i, 0)),
                pl.BlockSpec((bt, vt.shape[1]), lambda i: (i, 0))]
    in_specs += [spec(a) for a in wlist]
    return pl.pallas_call(
        _mlp_fc5_kernel,
        out_shape=jax.ShapeDtypeStruct((B, 16384), jnp.bfloat16),
        grid_spec=pltpu.PrefetchScalarGridSpec(
            num_scalar_prefetch=0,
            grid=(B // bt,),
            in_specs=in_specs,
            out_specs=pl.BlockSpec((bt, 16384), lambda i: (i, 0)),
        ),
        compiler_params=pltpu.CompilerParams(
            dimension_semantics=("parallel",),
            vmem_limit_bytes=_VMEM_LIMIT,
        ),
    )(*args)


def _zero_borders(S):
    bb, hp, wp, ch = S.shape
    S[:, 0:1, :, :] = jnp.zeros((bb, 1, wp, ch), S.dtype)
    S[:, hp - 1:hp, :, :] = jnp.zeros((bb, 1, wp, ch), S.dtype)
    S[:, :, 0:1, :] = jnp.zeros((bb, hp, 1, ch), S.dtype)
    S[:, :, wp - 1:wp, :] = jnp.zeros((bb, hp, 1, ch), S.dtype)


def _patch(S, taps, h, w, bb, cin):
    parts = [S[:, dy:dy + h, dx:dx + w, :] for (dy, dx) in taps]
    return jnp.concatenate(parts, axis=-1).reshape(bb * h * w, len(taps) * cin)


def _sx(S, h, w):
    return jnp.concatenate([S[:, :, dx:dx + w, :] for dx in range(3)], axis=-1)


def _rows_interleave(top, bot, bb, h, w2, cp):
    z = jnp.concatenate([top[:, :, None, :, :], bot[:, :, None, :, :]], axis=2)
    return z.reshape(bb, 2 * h, w2, cp)


def _decoder_kernel(x_ref,
                    u1w_ref, u1b_ref, c1w_ref, c1b_ref,
                    u2w_ref, u2b_ref, c2w_ref, c2b_ref,
                    u3w_ref, u3b_ref, c3w_ref, c3b_ref,
                    hw_ref, hb_ref,
                    o_ref,
                    S0, S1, S1b, S2, S2b, S3, S3b, *, bb):
    f32 = jnp.float32
    for S in (S0, S1, S1b, S2, S2b, S3, S3b):
        _zero_borders(S)

    S0[:, 1:9, 1:9, :] = x_ref[...]

    outs = []
    for ry in (0, 1):
        for rx in (0, 1):
            p = 2 * ry + rx
            taps = [(dy, dx) for dy in _tapsD(ry) for dx in _tapsD(rx)]
            pt = _patch(S0, taps, 8, 8, bb, 256)
            y = jnp.dot(pt, u1w_ref[p], preferred_element_type=f32)
            y = jnp.maximum(y + u1b_ref[p], 0.0).astype(jnp.bfloat16)
            outs.append(y.reshape(bb, 8, 8, 256))
    top = jnp.concatenate([outs[0][:, :, :, None, :], outs[1][:, :, :, None, :]],
                          axis=3).reshape(bb, 8, 16, 256)
    bot = jnp.concatenate([outs[2][:, :, :, None, :], outs[3][:, :, :, None, :]],
                          axis=3).reshape(bb, 8, 16, 256)
    S1[:, 1:17, 1:17, :] = _rows_interleave(top, bot, bb, 8, 16, 256)

    def full_patch(sx, h, w, kc):
        pt = jnp.concatenate([sx[:, dy:dy + h] for dy in range(3)], axis=-1)
        return pt.reshape(bb * h * w, kc)

    sx = _sx(S1, 16, 16)
    acc = jnp.dot(full_patch(sx, 16, 16, 2304), c1w_ref[...],
                  preferred_element_type=f32)
    acc = jnp.maximum(acc + c1b_ref[...], 0.0).astype(jnp.bfloat16)
    S1b[:, 1:17, 1:17, :] = acc.reshape(bb, 16, 16, 256)

    sx = _sx(S1b, 16, 16)
    rows = []
    for ry in (0, 1):
        d0, d1 = _tapsD(ry)
        pt = jnp.concatenate([sx[:, d0:d0 + 16], sx[:, d1:d1 + 16]],
                             axis=-1).reshape(bb * 256, 1536)
        acc = jnp.dot(pt, u2w_ref[ry], preferred_element_type=f32)
        y = jnp.maximum(acc + u2b_ref[ry], 0.0).astype(jnp.bfloat16)
        rows.append(y.reshape(bb, 16, 16, 2, 128).reshape(bb, 16, 32, 128))
    S2[:, 1:33, 1:33, :] = _rows_interleave(rows[0], rows[1], bb, 16, 32, 128)

    sx = _sx(S2, 32, 32)
    acc = jnp.dot(full_patch(sx, 32, 32, 1152), c2w_ref[...],
                  preferred_element_type=f32)
    acc = jnp.maximum(acc + c2b_ref[...], 0.0).astype(jnp.bfloat16)
    S2b[:, 1:33, 1:33, :] = acc.reshape(bb, 32, 32, 128)

    sx = _sx(S2b, 32, 32)
    acc = jnp.dot(full_patch(sx, 32, 32, 1152), u3w_ref[...],
                  preferred_element_type=f32)
    y = jnp.maximum(acc + u3b_ref[...], 0.0).astype(jnp.bfloat16)
    y = y.reshape(bb, 32, 32, 2, 2, 64)
    top = y[:, :, :, 0].reshape(bb, 32, 64, 64)
    bot = y[:, :, :, 1].reshape(bb, 32, 64, 64)
    S3[:, 1:65, 1:65, :] = _rows_interleave(top, bot, bb, 32, 64, 64)

    sx = _sx(S3, 64, 64)
    acc = jnp.dot(full_patch(sx, 64, 64, 576), c3w_ref[...],
                  preferred_element_type=f32)
    acc = jnp.maximum(acc + c3b_ref[...], 0.0).astype(jnp.bfloat16)
    S3b[:, 1:65, 1:65, :] = acc.reshape(bb, 64, 64, 64)

    sx = _sx(S3b, 64, 64)
    acc = jnp.dot(full_patch(sx, 64, 64, 576), hw_ref[...],
                  preferred_element_type=f32)
    acc = acc + hb_ref[...]
    lane = jax.lax.broadcasted_iota(jnp.int32, acc.shape, 1)
    acc = jnp.where(lane % 4 == 3, jax.nn.sigmoid(acc), acc)
    o_ref[...] = acc.reshape(bb, 64, 64, 16)


def _run_decoder(x, wlist, bb=4):
    B = x.shape[0]

    def spec(a):
        nd = a.ndim
        return pl.BlockSpec(a.shape, lambda i, nd=nd: (0,) * nd)

    bf = jnp.bfloat16
    scratch = [
        pltpu.VMEM((bb, 10, 10, 256), bf),
        pltpu.VMEM((bb, 18, 18, 256), bf),
        pltpu.VMEM((bb, 18, 18, 256), bf),
        pltpu.VMEM((bb, 34, 34, 128), bf),
        pltpu.VMEM((bb, 34, 34, 128), bf),
        pltpu.VMEM((bb, 66, 66, 64), bf),
        pltpu.VMEM((bb, 66, 66, 64), bf),
    ]
    import functools
    return pl.pallas_call(
        functools.partial(_decoder_kernel, bb=bb),
        out_shape=jax.ShapeDtypeStruct((B, 64, 64, 16), jnp.float32),
        grid_spec=pltpu.PrefetchScalarGridSpec(
            num_scalar_prefetch=0,
            grid=(B // bb,),
            in_specs=[pl.BlockSpec((bb, 8, 8, 256), lambda i: (i, 0, 0, 0))]
                     + [spec(a) for a in wlist],
            out_specs=pl.BlockSpec((bb, 64, 64, 16), lambda i: (i, 0, 0, 0)),
            scratch_shapes=scratch,
        ),
        compiler_params=pltpu.CompilerParams(
            dimension_semantics=("parallel",),
            vmem_limit_bytes=_VMEM_LIMIT,
        ),
    )(x, *wlist)


def _prep_up_parity(w, b, cp):
    ws, bs = [], []
    for ry in (0, 1):
        for rx in (0, 1):
            p = 2 * ry + rx
            blocks = [w[3 * dy + dx, :, p * cp:(p + 1) * cp]
                      for dy in _tapsD(ry) for dx in _tapsD(rx)]
            ws.append(jnp.concatenate(blocks, 0))
            bs.append(b[:, p * cp:(p + 1) * cp])
    return jnp.stack(ws), jnp.stack(bs)


def _prep_up_pair(w, b, cp):
    ws, bs = [], []
    for ry in (0, 1):
        rows = []
        for dy in _tapsD(ry):
            for dx in range(3):
                cols = []
                for rx in (0, 1):
                    blk = w[3 * dy + dx, :, (2 * ry + rx) * cp:(2 * ry + rx + 1) * cp]
                    if dx not in _tapsD(rx):
                        blk = jnp.zeros_like(blk)
                    cols.append(blk)
                rows.append(jnp.concatenate(cols, 1))
        ws.append(jnp.concatenate(rows, 0))
        bs.append(jnp.concatenate([b[:, 2 * ry * cp:(2 * ry + 1) * cp],
                                   b[:, (2 * ry + 1) * cp:(2 * ry + 2) * cp]], 1))
    return jnp.stack(ws), jnp.stack(bs)


def _prep_up_quad(w, b, cp_in, cp_out):
    rows = []
    for tap in range(9):
        dy, dx = tap // 3, tap % 3
        cols = []
        for ry in (0, 1):
            for rx in (0, 1):
                blk = w[tap, :, (2 * ry + rx) * cp_in:(2 * ry + rx) * cp_in + cp_out]
                if (dy not in _tapsD(ry)) or (dx not in _tapsD(rx)):
                    blk = jnp.zeros_like(blk)
                cols.append(blk)
        rows.append(jnp.concatenate(cols, 1))
    wq = jnp.concatenate(rows, 0)
    bq = jnp.concatenate([b[:, p * cp_in:p * cp_in + cp_out] for p in range(4)], 1)
    return wq, bq


def kernel(mlp_00, mlp_01, mlp_02, mlp_03, mlp_04, mlp_05, mlp_06, mlp_07,
           mlp_08, mlp_09, mlp_10, mlp_11, mlp_12, mlp_13, mlp_14, mlp_15,
           fc5_w, fc5_b,
           upconv1_w, upconv1_b, conv1_1_w, conv1_1_b,
           upconv2_w, upconv2_b, conv2_1_w, conv2_1_b,
           upconv3_w, upconv3_b, conv3_1_w, conv3_1_b,
           head_w, head_b,
           c, v, t):
    bf = jnp.bfloat16
    B = c.shape[0]

    w1c, b1c, w2c, b2c = mlp_00, mlp_01, mlp_02, mlp_03
    w1v, b1v, w2v, b2v = mlp_04, mlp_05, mlp_06, mlp_07
    w1t, b1t, w2t, b2t = mlp_08, mlp_09, mlp_10, mlp_11
    w3, b3, w4, b4 = mlp_12, mlp_13, mlp_14, mlp_15

    z44 = jnp.zeros((4, 512), bf)
    z12 = jnp.zeros((12, 512), bf)
    w1vt = jnp.concatenate(
        [jnp.concatenate([w1v, z44], 1), jnp.concatenate([z12, w1t], 1)], 0)
    b1vt = jnp.concatenate([b1v, b1t], 1)
    z5 = jnp.zeros((512, 512), bf)
    w2vt = jnp.concatenate(
        [jnp.concatenate([w2v, z5], 1), jnp.concatenate([z5, w2t], 1)], 0)
    b2vt = jnp.concatenate([b2v, b2t], 1)
    w3c, w3vt = w3[0:512], w3[512:1536]

    vt = jnp.concatenate([v, t], 1).astype(bf)
    mlp_wlist = [w1c, b1c, w2c, b2c, w1vt, b1vt, w2vt, b2vt,
                 w3c, w3vt, b3, w4, b4, fc5_w, fc5_b]
    bt = 128 if B % 128 == 0 else B
    x = _run_mlp_fc5(c.astype(bf), vt, mlp_wlist, bt=bt)
    x = x.reshape(B, 8, 8, 256)

    def rowform(w9):
        return jnp.stack([jnp.concatenate([w9[3 * dy + dx] for dx in range(3)], 0)
                          for dy in range(3)])

    u1w, u1b = _prep_up_parity(upconv1_w, upconv1_b, 256)
    c1w = rowform(conv1_1_w).reshape(2304, 256)
    u2w, u2b = _prep_up_pair(upconv2_w, upconv2_b, 128)
    c2w = rowform(conv2_1_w).reshape(1152, 128)
    u3w, u3b = _prep_up_quad(upconv3_w, upconv3_b, 128, 64)
    c3w = rowform(conv3_1_w[:, :64, :64]).reshape(576, 64)
    c3b = conv3_1_b[:, :64]
    hw16 = jnp.concatenate([head_w[:, :64, p * 32:p * 32 + 4] for p in range(4)], 2)
    hw = rowform(hw16).reshape(576, 16)
    hb = jnp.concatenate([head_b[:, p * 32:p * 32 + 4] for p in range(4)], 1)

    dec_wlist = [u1w, u1b, c1w, conv1_1_b, u2w, u2b, c2w, conv2_1_b,
                 u3w, u3b, c3w, c3b, hw, hb]
    bb = 2 if B % 2 == 0 else B
    head = _run_decoder(x, dec_wlist, bb=bb)

    r = head.reshape(B, 64, 64, 2, 2, 4)
    r = r.transpose(0, 5, 1, 3, 2, 4).reshape(B, 4, 128, 128)
    return r[:, 0:3], r[:, 3:4]
